# Initial kernel scaffold; baseline (speedup 1.0000x reference)
import jax
import jax.numpy as jnp
from jax.experimental import pallas as pl


def kernel(x, edge_index, W, b):
    raise NotImplementedError("write your pallas kernel here")


# speedup vs baseline: 1.4512x; 1.4512x over previous
import functools

import jax
import jax.numpy as jnp
from jax import lax
from jax.experimental import pallas as pl
from jax.experimental.pallas import tpu as pltpu
from jax.experimental.pallas import tpu_sc as plsc

_B = 2
_C = 128
_N = 10000
_K = 16
_OUT_C = 128
_G = 4

_NW = 32
_NODES = _B * _N
_CHUNK = 8
_NODES_PAD = 20480
_NPW = _NODES_PAD // _NW
_CPW = _NPW // _CHUNK
_IPW = _NPW * _K


def _sc_max_relative(xt, idx0, idx1):
    mesh = plsc.VectorSubcoreMesh(core_axis_name="c", subcore_axis_name="s")

    @functools.partial(
        pl.kernel,
        mesh=mesh,
        out_type=jax.ShapeDtypeStruct((_NODES_PAD, _C), jnp.float32),
        scratch_types=[
            pltpu.VMEM((_IPW,), jnp.int32),
            pltpu.VMEM((_IPW,), jnp.int32),
            pltpu.VMEM((_CHUNK * _K, _C), jnp.float32),
            pltpu.VMEM((_CHUNK * _K, _C), jnp.float32),
            pltpu.VMEM((_CHUNK, _C), jnp.float32),
            pltpu.SemaphoreType.DMA,
            pltpu.SemaphoreType.DMA,
        ],
    )
    def k(xt_hbm, i0_hbm, i1_hbm, out_hbm, i0_v, i1_v, r0_v, r1_v, ob_v,
          sem0, sem1):
        wid = lax.axis_index("s") * 2 + lax.axis_index("c")
        node_base = wid * _NPW
        idx_base = wid * _IPW
        pltpu.sync_copy(i0_hbm.at[pl.ds(idx_base, _IPW)], i0_v)
        pltpu.sync_copy(i1_hbm.at[pl.ds(idx_base, _IPW)], i1_v)

        def chunk_body(ci, carry):
            off = ci * (_CHUNK * _K)
            cp0 = pltpu.async_copy(
                xt_hbm.at[i0_v.at[pl.ds(off, _CHUNK * _K)]], r0_v, sem0)
            cp1 = pltpu.async_copy(
                xt_hbm.at[i1_v.at[pl.ds(off, _CHUNK * _K)]], r1_v, sem1)
            cp0.wait()
            cp1.wait()

            def node_body(nj, c2):
                row = nj * _K
                for cc in range(_C // 16):
                    sl = pl.ds(cc * 16, 16)
                    acc = r0_v[row, sl] - r1_v[row, sl]
                    for kk in range(1, _K):
                        acc = jnp.maximum(
                            acc, r0_v[row + kk, sl] - r1_v[row + kk, sl])
                    ob_v[nj, sl] = acc
                return c2

            lax.fori_loop(0, _CHUNK, node_body, 0)
            pltpu.sync_copy(
                ob_v, out_hbm.at[pl.ds(node_base + ci * _CHUNK, _CHUNK)])
            return carry

        lax.fori_loop(0, _CPW, chunk_body, 0)

    return k(xt, idx0, idx1)


def _tc_conv_kernel(xt_ref, xj_ref, wx_ref, wj_ref, b_ref, o_ref):
    o = jnp.dot(xt_ref[...], wx_ref[...], preferred_element_type=jnp.float32)
    o = o + jnp.dot(xj_ref[...], wj_ref[...],
                    preferred_element_type=jnp.float32)
    o_ref[...] = jnp.maximum(o + b_ref[...], 0.0)


def _tc_conv(xt, xj, wxT, wjT, bias):
    rows = xt.shape[0]
    blk = 2000
    grid = rows // blk
    return pl.pallas_call(
        _tc_conv_kernel,
        grid=(grid,),
        in_specs=[
            pl.BlockSpec((blk, _C), lambda i: (i, 0)),
            pl.BlockSpec((blk, _C), lambda i: (i, 0)),
            pl.BlockSpec((_C, _OUT_C), lambda i: (0, 0)),
            pl.BlockSpec((_C, _OUT_C), lambda i: (0, 0)),
            pl.BlockSpec((1, _OUT_C), lambda i: (0, 0)),
        ],
        out_specs=pl.BlockSpec((blk, _OUT_C), lambda i: (i, 0)),
        out_shape=jax.ShapeDtypeStruct((rows, _OUT_C), jnp.float32),
    )(xt, xj, wxT, wjT, bias)


def kernel(x, edge_index, W, b):
    xt = jnp.transpose(x[..., 0], (0, 2, 1)).reshape(_NODES, _C)

    boff = (jnp.arange(_B, dtype=jnp.int32) * _N)[:, None, None]
    pad = jnp.zeros((_NODES_PAD * _K - _NODES * _K,), jnp.int32)
    idx0 = jnp.concatenate([(edge_index[0] + boff).reshape(-1), pad])
    idx1 = jnp.concatenate([(edge_index[1] + boff).reshape(-1), pad])

    xj = _sc_max_relative(xt, idx0, idx1)[:_NODES]

    gsz_o = _OUT_C // _G
    gsz_i = _C // _G
    Wx_p = W[:, 0::2].reshape(_G, gsz_o, gsz_i)
    Wj_p = W[:, 1::2].reshape(_G, gsz_o, gsz_i)
    eye = jnp.eye(_G, dtype=W.dtype)
    Wx_bd = jnp.einsum('goc,gh->gohc', Wx_p, eye).reshape(_OUT_C, _C)
    Wj_bd = jnp.einsum('goc,gh->gohc', Wj_p, eye).reshape(_OUT_C, _C)

    out_nm = _tc_conv(xt, xj, Wx_bd.T, Wj_bd.T, b.reshape(1, _OUT_C))

    out = jnp.transpose(out_nm.reshape(_B, _N, _OUT_C), (0, 2, 1))
    return out[..., None]


# speedup vs baseline: 1.6297x; 1.1230x over previous
import functools

import jax
import jax.numpy as jnp
from jax import lax
from jax.experimental import pallas as pl
from jax.experimental.pallas import tpu as pltpu
from jax.experimental.pallas import tpu_sc as plsc

_B = 2
_C = 128
_N = 10000
_K = 16
_OUT_C = 128
_G = 4

_NW = 32
_NODES = _B * _N
_CHUNK = 8
_NODES_PAD = 20480
_NPW = _NODES_PAD // _NW
_CPW = _NPW // _CHUNK
_IPW = _NPW * _K


def _sc_max_relative(xt, idx0, idx1):
    mesh = plsc.VectorSubcoreMesh(core_axis_name="c", subcore_axis_name="s")
    _CK = _CHUNK * _K

    @functools.partial(
        pl.kernel,
        mesh=mesh,
        out_type=jax.ShapeDtypeStruct((_NODES_PAD, _C), jnp.float32),
        scratch_types=[
            pltpu.VMEM((_IPW + _CK,), jnp.int32),
            pltpu.VMEM((_IPW + _CK,), jnp.int32),
            pltpu.VMEM((_CK, _C), jnp.float32),
            pltpu.VMEM((_CK, _C), jnp.float32),
            pltpu.VMEM((_CK, _C), jnp.float32),
            pltpu.VMEM((_CK, _C), jnp.float32),
            pltpu.VMEM((_CHUNK, _C), jnp.float32),
            pltpu.SemaphoreType.DMA,
            pltpu.SemaphoreType.DMA,
        ],
    )
    def k(xt_hbm, i0_hbm, i1_hbm, out_hbm, i0_v, i1_v, r0a, r1a, r0b, r1b,
          ob_v, semA, semB):
        wid = lax.axis_index("s") * 2 + lax.axis_index("c")
        node_base = wid * _NPW
        idx_base = wid * _IPW
        pltpu.sync_copy(i0_hbm.at[pl.ds(idx_base, _IPW)],
                        i0_v.at[pl.ds(0, _IPW)])
        pltpu.sync_copy(i1_hbm.at[pl.ds(idx_base, _IPW)],
                        i1_v.at[pl.ds(0, _IPW)])
        pltpu.sync_copy(i0_hbm.at[pl.ds(idx_base, _CK)],
                        i0_v.at[pl.ds(_IPW, _CK)])
        pltpu.sync_copy(i1_hbm.at[pl.ds(idx_base, _CK)],
                        i1_v.at[pl.ds(_IPW, _CK)])

        def start(ci, r0, r1, sem):
            off = ci * _CK
            pltpu.async_copy(xt_hbm.at[i0_v.at[pl.ds(off, _CK)]], r0, sem)
            pltpu.async_copy(xt_hbm.at[i1_v.at[pl.ds(off, _CK)]], r1, sem)

        def wait_pair(r0, r1, sem):
            pltpu.make_async_copy(xt_hbm.at[pl.ds(0, _CK)], r0, sem).wait()
            pltpu.make_async_copy(xt_hbm.at[pl.ds(0, _CK)], r1, sem).wait()

        def compute(ci, r0, r1):
            def node_body(nj, c2):
                row = nj * _K
                for cc in range(_C // 16):
                    sl = pl.ds(cc * 16, 16)
                    acc = r0[row, sl] - r1[row, sl]
                    for kk in range(1, _K):
                        acc = jnp.maximum(
                            acc, r0[row + kk, sl] - r1[row + kk, sl])
                    ob_v[nj, sl] = acc
                return c2

            lax.fori_loop(0, _CHUNK, node_body, 0)
            pltpu.sync_copy(
                ob_v, out_hbm.at[pl.ds(node_base + ci * _CHUNK, _CHUNK)])

        start(0, r0a, r1a, semA)

        def body2(i2, carry):
            ci = i2 * 2
            start(ci + 1, r0b, r1b, semB)
            wait_pair(r0a, r1a, semA)
            compute(ci, r0a, r1a)
            start(ci + 2, r0a, r1a, semA)
            wait_pair(r0b, r1b, semB)
            compute(ci + 1, r0b, r1b)
            return carry

        lax.fori_loop(0, _CPW // 2, body2, 0)
        wait_pair(r0a, r1a, semA)

    return k(xt, idx0, idx1)


def _tc_conv_kernel(xt_ref, xj_ref, wx_ref, wj_ref, b_ref, o_ref):
    o = jnp.dot(xt_ref[...], wx_ref[...], preferred_element_type=jnp.float32)
    o = o + jnp.dot(xj_ref[...], wj_ref[...],
                    preferred_element_type=jnp.float32)
    o_ref[...] = jnp.maximum(o + b_ref[...], 0.0)


def _tc_conv(xt, xj, wxT, wjT, bias):
    rows = xt.shape[0]
    blk = 2000
    grid = rows // blk
    return pl.pallas_call(
        _tc_conv_kernel,
        grid=(grid,),
        in_specs=[
            pl.BlockSpec((blk, _C), lambda i: (i, 0)),
            pl.BlockSpec((blk, _C), lambda i: (i, 0)),
            pl.BlockSpec((_C, _OUT_C), lambda i: (0, 0)),
            pl.BlockSpec((_C, _OUT_C), lambda i: (0, 0)),
            pl.BlockSpec((1, _OUT_C), lambda i: (0, 0)),
        ],
        out_specs=pl.BlockSpec((blk, _OUT_C), lambda i: (i, 0)),
        out_shape=jax.ShapeDtypeStruct((rows, _OUT_C), jnp.float32),
    )(xt, xj, wxT, wjT, bias)


def kernel(x, edge_index, W, b):
    xt = jnp.transpose(x[..., 0], (0, 2, 1)).reshape(_NODES, _C)

    boff = (jnp.arange(_B, dtype=jnp.int32) * _N)[:, None, None]
    pad = jnp.zeros((_NODES_PAD * _K - _NODES * _K,), jnp.int32)
    idx0 = jnp.concatenate([(edge_index[0] + boff).reshape(-1), pad])
    idx1 = jnp.concatenate([(edge_index[1] + boff).reshape(-1), pad])

    xj = _sc_max_relative(xt, idx0, idx1)[:_NODES]

    gsz_o = _OUT_C // _G
    gsz_i = _C // _G
    Wx_p = W[:, 0::2].reshape(_G, gsz_o, gsz_i)
    Wj_p = W[:, 1::2].reshape(_G, gsz_o, gsz_i)
    eye = jnp.eye(_G, dtype=W.dtype)
    Wx_bd = jnp.einsum('goc,gh->gohc', Wx_p, eye).reshape(_OUT_C, _C)
    Wj_bd = jnp.einsum('goc,gh->gohc', Wj_p, eye).reshape(_OUT_C, _C)

    out_nm = _tc_conv(xt, xj, Wx_bd.T, Wj_bd.T, b.reshape(1, _OUT_C))

    out = jnp.transpose(out_nm.reshape(_B, _N, _OUT_C), (0, 2, 1))
    return out[..., None]
